# baseline (speedup 1.0000x reference)
import jax
import jax.numpy as jnp
from jax.experimental import pallas as pl
from jax.experimental.pallas import tpu as pltpu

_CHUNK = 1
_DEPTH = 8


def _pool_body(x_hbm, c_ref, w_ref, b_ref, o_ref, bufs, sems):
    B = x_hbm.shape[0]
    n_chunks = B // _CHUNK
    w = w_ref[...]
    c = c_ref[...]
    bvec = b_ref[...]

    def dma_in(slot, chunk):
        pltpu.make_async_copy(
            x_hbm.at[pl.ds(chunk * _CHUNK, _CHUNK)], bufs.at[slot], sems.at[slot]
        ).start()

    for i in range(_DEPTH):
        dma_in(i, i)

    def body(chunk, _):
        slot = jax.lax.rem(chunk, _DEPTH)
        pltpu.make_async_copy(bufs.at[slot], bufs.at[slot], sems.at[slot]).wait()
        for g in range(_CHUNK):
            x = bufs[slot, g]
            logits = jax.lax.dot_general(
                w, x, (((1,), (1,)), ((), ())), preferred_element_type=jnp.float32
            )
            logits = logits + bvec
            m = jnp.max(logits, axis=0, keepdims=True)
            e = jnp.exp(logits - m)
            s = jnp.sum(e, axis=0, keepdims=True)
            a = e / s
            ax = jax.lax.dot_general(
                a, x, (((1,), (0,)), ((), ())), preferred_element_type=jnp.float32
            )
            asum = jnp.sum(a, axis=1, keepdims=True)
            pooled = ax - asum * c
            ss = jnp.sum(pooled * pooled, axis=1, keepdims=True)
            ss = jnp.sum(ss, axis=0, keepdims=True)
            norm = jnp.maximum(jnp.sqrt(ss), 1e-12)
            o_ref[chunk * _CHUNK + g] = pooled / norm

        @pl.when(chunk + _DEPTH < n_chunks)
        def _():
            dma_in(slot, chunk + _DEPTH)

        return ()

    jax.lax.fori_loop(0, n_chunks, body, ())


def kernel(x, centers, attn_w, attn_b):
    B, T, D = x.shape
    K = centers.shape[0]
    out = pl.pallas_call(
        _pool_body,
        out_shape=jax.ShapeDtypeStruct((B, K, D), x.dtype),
        in_specs=[
            pl.BlockSpec(memory_space=pl.ANY),
            pl.BlockSpec((K, D), lambda: (0, 0)),
            pl.BlockSpec((K, D), lambda: (0, 0)),
            pl.BlockSpec((K, 1), lambda: (0, 0)),
        ],
        out_specs=pl.BlockSpec((B, K, D), lambda: (0, 0, 0)),
        scratch_shapes=[
            pltpu.VMEM((_DEPTH, _CHUNK, T, D), jnp.float32),
            pltpu.SemaphoreType.DMA((_DEPTH,)),
        ],
        compiler_params=pltpu.CompilerParams(
            vmem_limit_bytes=48 * 1024 * 1024,
        ),
        name="temporal_pooling",
    )(x, centers, attn_w, attn_b.reshape(K, 1))
    return out.reshape(B, K * D)


# speedup vs baseline: 1.0714x; 1.0714x over previous
import jax
import jax.numpy as jnp
from jax.experimental import pallas as pl
from jax.experimental.pallas import tpu as pltpu

_CHUNK = 1
_DEPTH = 3


def _pool_body(x_hbm, c_ref, w_ref, b_ref, o_ref, bufs, sems):
    B = x_hbm.shape[0]
    n_chunks = B // _CHUNK
    w = w_ref[...]
    c = c_ref[...]
    bvec = b_ref[...]

    def dma_in(slot, chunk):
        pltpu.make_async_copy(
            x_hbm.at[pl.ds(chunk * _CHUNK, _CHUNK)], bufs.at[slot], sems.at[slot]
        ).start()

    for i in range(_DEPTH):
        dma_in(i, i)

    def body(chunk, _):
        slot = jax.lax.rem(chunk, _DEPTH)
        pltpu.make_async_copy(bufs.at[slot], bufs.at[slot], sems.at[slot]).wait()
        for g in range(_CHUNK):
            x = bufs[slot, g]
            logits = jax.lax.dot_general(
                w, x, (((1,), (1,)), ((), ())), preferred_element_type=jnp.float32
            )
            logits = logits + bvec
            m = jnp.max(logits, axis=0, keepdims=True)
            e = jnp.exp(logits - m)
            s = jnp.sum(e, axis=0, keepdims=True)
            a = e / s
            ax = jax.lax.dot_general(
                a, x, (((1,), (0,)), ((), ())), preferred_element_type=jnp.float32
            )
            asum = jnp.sum(a, axis=1, keepdims=True)
            pooled = ax - asum * c
            ss = jnp.sum(pooled * pooled, axis=1, keepdims=True)
            ss = jnp.sum(ss, axis=0, keepdims=True)
            norm = jnp.maximum(jnp.sqrt(ss), 1e-12)
            o_ref[chunk * _CHUNK + g] = pooled / norm

        @pl.when(chunk + _DEPTH < n_chunks)
        def _():
            dma_in(slot, chunk + _DEPTH)

        return ()

    jax.lax.fori_loop(0, n_chunks, body, ())


def kernel(x, centers, attn_w, attn_b):
    B, T, D = x.shape
    K = centers.shape[0]
    out = pl.pallas_call(
        _pool_body,
        out_shape=jax.ShapeDtypeStruct((B, K, D), x.dtype),
        in_specs=[
            pl.BlockSpec(memory_space=pl.ANY),
            pl.BlockSpec((K, D), lambda: (0, 0)),
            pl.BlockSpec((K, D), lambda: (0, 0)),
            pl.BlockSpec((K, 1), lambda: (0, 0)),
        ],
        out_specs=pl.BlockSpec((B, K, D), lambda: (0, 0, 0)),
        scratch_shapes=[
            pltpu.VMEM((_DEPTH, _CHUNK, T, D), jnp.float32),
            pltpu.SemaphoreType.DMA((_DEPTH,)),
        ],
        compiler_params=pltpu.CompilerParams(
            vmem_limit_bytes=48 * 1024 * 1024,
        ),
        name="temporal_pooling",
    )(x, centers, attn_w, attn_b.reshape(K, 1))
    return out.reshape(B, K * D)
